# baseline (speedup 1.0000x reference)
import functools

import jax
import jax.numpy as jnp
from jax import lax
from jax.experimental import pallas as pl
from jax.experimental.pallas import tpu as pltpu
from jax.experimental.pallas import tpu_sc as plsc

L = 2
M = 100000
H = 128
B = 16384

NC = 2
NS = 16
NW = NC * NS

COPY_BLOCK = 4000
N_BLOCKS = (L * M) // COPY_BLOCK

RB_PER_W = (L * B) // NW
CHUNK = 128
N_CH = RB_PER_W // CHUNK


N_DMA = 8


def _tc_copy_body(hsrc, csrc, hdst, cdst, *sems):
  rows = (L * M) // N_DMA
  copies = []
  for a, (src, dst) in enumerate(((hsrc, hdst), (csrc, cdst))):
    for i in range(N_DMA):
      copies.append(pltpu.make_async_copy(
          src.at[pl.ds(i * rows, rows)],
          dst.at[pl.ds(i * rows, rows)],
          sems[a * N_DMA + i]))
  for cp in copies:
    cp.start()
  for cp in copies:
    cp.wait()


_tc_copy = pl.pallas_call(
    _tc_copy_body,
    in_specs=[
        pl.BlockSpec(memory_space=pl.ANY),
        pl.BlockSpec(memory_space=pl.ANY),
    ],
    out_specs=[
        pl.BlockSpec(memory_space=pl.ANY),
        pl.BlockSpec(memory_space=pl.ANY),
    ],
    out_shape=[
        jax.ShapeDtypeStruct((L * M, H), jnp.float32),
        jax.ShapeDtypeStruct((L * M, H), jnp.float32),
    ],
    scratch_shapes=[pltpu.SemaphoreType.DMA] * (2 * N_DMA),
)


def _sc_gather_body(memh, memc, idx2, hin, cin,
                    gh0, gh1, gc0, gc1, ib0, ib1,
                    sgh0, sgh1, sgc0, sgc1, swh0, swh1, swc0, swc1):
  c = lax.axis_index("c")
  s = lax.axis_index("s")
  base0 = (c * NS + s) * RB_PER_W
  gh = (gh0, gh1)
  gc = (gc0, gc1)
  ib = (ib0, ib1)
  sgh = (sgh0, sgh1)
  sgc = (sgc0, sgc1)
  swh = (swh0, swh1)
  swc = (swc0, swc1)

  def fetch(j, b):
    base = base0 + j * CHUNK
    pltpu.sync_copy(idx2.at[pl.ds(base, CHUNK)], ib[b])
    pltpu.async_copy(memh.at[ib[b]], gh[b], sgh[b])
    pltpu.async_copy(memc.at[ib[b]], gc[b], sgc[b])

  fetch(0, 0)
  for t in range(N_CH):
    b = t % 2
    nb = 1 - b
    if t + 1 < N_CH:
      if t >= 1:
        pltpu.make_async_copy(gh[nb], hin.at[pl.ds(0, CHUNK)], swh[nb]).wait()
        pltpu.make_async_copy(gc[nb], cin.at[pl.ds(0, CHUNK)], swc[nb]).wait()
      fetch(t + 1, nb)
    base = base0 + t * CHUNK
    pltpu.make_async_copy(memh.at[pl.ds(0, CHUNK)], gh[b], sgh[b]).wait()
    pltpu.async_copy(gh[b], hin.at[pl.ds(base, CHUNK)], swh[b])
    pltpu.make_async_copy(memc.at[pl.ds(0, CHUNK)], gc[b], sgc[b]).wait()
    pltpu.async_copy(gc[b], cin.at[pl.ds(base, CHUNK)], swc[b])
  for b in range(2):
    pltpu.make_async_copy(gh[b], hin.at[pl.ds(0, CHUNK)], swh[b]).wait()
    pltpu.make_async_copy(gc[b], cin.at[pl.ds(0, CHUNK)], swc[b]).wait()


_sc_gather = functools.partial(
    pl.kernel,
    out_type=(
        jax.ShapeDtypeStruct((L * B, H), jnp.float32),
        jax.ShapeDtypeStruct((L * B, H), jnp.float32),
    ),
    mesh=plsc.VectorSubcoreMesh(core_axis_name="c", subcore_axis_name="s"),
    scratch_types=(
        [pltpu.VMEM((CHUNK, H), jnp.float32) for _ in range(4)]
        + [pltpu.VMEM((CHUNK,), jnp.int32) for _ in range(2)]
        + [pltpu.SemaphoreType.DMA for _ in range(8)]
    ),
)(_sc_gather_body)


def _sc_scatter_body(hv, cv, idx2, src2, outh, outc,
                     sh0, sh1, sc0, sc1, ib0, ib1, sb0, sb1,
                     sgh0, sgh1, sgc0, sgc1, swh0, swh1, swc0, swc1):
  c = lax.axis_index("c")
  s = lax.axis_index("s")
  base0 = (c * NS + s) * RB_PER_W
  sh = (sh0, sh1)
  sc = (sc0, sc1)
  ib = (ib0, ib1)
  sb = (sb0, sb1)
  sgh = (sgh0, sgh1)
  sgc = (sgc0, sgc1)
  swh = (swh0, swh1)
  swc = (swc0, swc1)

  def fetch(j, b):
    base = base0 + j * CHUNK
    pltpu.sync_copy(idx2.at[pl.ds(base, CHUNK)], ib[b])
    pltpu.sync_copy(src2.at[pl.ds(base, CHUNK)], sb[b])
    pltpu.async_copy(hv.at[sb[b]], sh[b], sgh[b])
    pltpu.async_copy(cv.at[sb[b]], sc[b], sgc[b])

  fetch(0, 0)
  for t in range(N_CH):
    b = t % 2
    nb = 1 - b
    if t + 1 < N_CH:
      if t >= 1:
        pltpu.make_async_copy(sh[nb], outh.at[pl.ds(0, CHUNK)], swh[nb]).wait()
        pltpu.make_async_copy(sc[nb], outc.at[pl.ds(0, CHUNK)], swc[nb]).wait()
      fetch(t + 1, nb)
    pltpu.make_async_copy(hv.at[pl.ds(0, CHUNK)], sh[b], sgh[b]).wait()
    pltpu.async_copy(sh[b], outh.at[ib[b]], swh[b])
    pltpu.make_async_copy(cv.at[pl.ds(0, CHUNK)], sc[b], sgc[b]).wait()
    pltpu.async_copy(sc[b], outc.at[ib[b]], swc[b])
  for b in range(2):
    pltpu.make_async_copy(sh[b], outh.at[pl.ds(0, CHUNK)], swh[b]).wait()
    pltpu.make_async_copy(sc[b], outc.at[pl.ds(0, CHUNK)], swc[b]).wait()


_sc_scatter = functools.partial(
    pl.kernel,
    out_type=(),
    mesh=plsc.VectorSubcoreMesh(core_axis_name="c", subcore_axis_name="s"),
    scratch_types=(
        [pltpu.VMEM((CHUNK, H), jnp.float32) for _ in range(4)]
        + [pltpu.VMEM((CHUNK,), jnp.int32) for _ in range(4)]
        + [pltpu.SemaphoreType.DMA for _ in range(8)]
    ),
)(_sc_scatter_body)


def kernel(mem_h, mem_c, slots, h_out, c_out):
  slots = slots.astype(jnp.int32)
  iota = lax.iota(jnp.int32, B)
  ss, si = lax.sort((slots, iota), num_keys=2)
  rs = ss[::-1]
  is_start = jnp.concatenate([jnp.ones((1,), jnp.bool_), rs[1:] != rs[:-1]])
  posr = lax.cummax(jnp.where(is_start, iota, 0))
  win = si[::-1][posr][::-1]
  gidx = jnp.concatenate([slots, slots + M])
  sidx = jnp.concatenate([ss, ss + M])
  ssrc = jnp.concatenate([win, win + B])

  hin, cin = _sc_gather(
      mem_h.reshape(L * M, H), mem_c.reshape(L * M, H), gidx)
  outh0, outc0 = _tc_copy(mem_h.reshape(L * M, H), mem_c.reshape(L * M, H))
  rh = jax.new_ref(outh0)
  rc = jax.new_ref(outc0)
  _sc_scatter(
      h_out.reshape(L * B, H),
      c_out.reshape(L * B, H),
      sidx,
      ssrc,
      rh,
      rc,
  )
  return (hin.reshape(L, B, H), cin.reshape(L, B, H),
          rh[...].reshape(L, M, H), rc[...].reshape(L, M, H))


# speedup vs baseline: 26.6464x; 26.6464x over previous
import functools

import jax
import jax.numpy as jnp
from jax import lax
from jax.experimental import pallas as pl
from jax.experimental.pallas import tpu as pltpu
from jax.experimental.pallas import tpu_sc as plsc

L = 2
M = 100000
H = 128
B = 16384

NC = 2
NS = 16
NW = NC * NS

COPY_BLOCK = 4000
N_BLOCKS = (L * M) // COPY_BLOCK

RB_PER_W = (L * B) // NW
CHUNK = 128
N_CH = RB_PER_W // CHUNK


def _tc_copy_body(hsrc, csrc, hdst, cdst):
  hdst[...] = hsrc[...]
  cdst[...] = csrc[...]


_tc_copy = pl.pallas_call(
    _tc_copy_body,
    grid=(N_BLOCKS,),
    in_specs=[
        pl.BlockSpec((COPY_BLOCK, H), lambda i: (i, 0)),
        pl.BlockSpec((COPY_BLOCK, H), lambda i: (i, 0)),
    ],
    out_specs=[
        pl.BlockSpec((COPY_BLOCK, H), lambda i: (i, 0)),
        pl.BlockSpec((COPY_BLOCK, H), lambda i: (i, 0)),
    ],
    out_shape=[
        jax.ShapeDtypeStruct((L * M, H), jnp.float32),
        jax.ShapeDtypeStruct((L * M, H), jnp.float32),
    ],
)


def _sc_gather_body(memh, memc, idx2, hin, cin,
                    gh0, gh1, gc0, gc1, ib0, ib1,
                    sgh0, sgh1, sgc0, sgc1, swh0, swh1, swc0, swc1):
  c = lax.axis_index("c")
  s = lax.axis_index("s")
  base0 = (c * NS + s) * RB_PER_W
  gh = (gh0, gh1)
  gc = (gc0, gc1)
  ib = (ib0, ib1)
  sgh = (sgh0, sgh1)
  sgc = (sgc0, sgc1)
  swh = (swh0, swh1)
  swc = (swc0, swc1)

  def fetch(j, b):
    base = base0 + j * CHUNK
    pltpu.sync_copy(idx2.at[pl.ds(base, CHUNK)], ib[b])
    pltpu.async_copy(memh.at[ib[b]], gh[b], sgh[b])
    pltpu.async_copy(memc.at[ib[b]], gc[b], sgc[b])

  fetch(0, 0)
  for t in range(N_CH):
    b = t % 2
    nb = 1 - b
    if t + 1 < N_CH:
      if t >= 1:
        pltpu.make_async_copy(gh[nb], hin.at[pl.ds(0, CHUNK)], swh[nb]).wait()
        pltpu.make_async_copy(gc[nb], cin.at[pl.ds(0, CHUNK)], swc[nb]).wait()
      fetch(t + 1, nb)
    base = base0 + t * CHUNK
    pltpu.make_async_copy(memh.at[pl.ds(0, CHUNK)], gh[b], sgh[b]).wait()
    pltpu.async_copy(gh[b], hin.at[pl.ds(base, CHUNK)], swh[b])
    pltpu.make_async_copy(memc.at[pl.ds(0, CHUNK)], gc[b], sgc[b]).wait()
    pltpu.async_copy(gc[b], cin.at[pl.ds(base, CHUNK)], swc[b])
  for b in range(2):
    pltpu.make_async_copy(gh[b], hin.at[pl.ds(0, CHUNK)], swh[b]).wait()
    pltpu.make_async_copy(gc[b], cin.at[pl.ds(0, CHUNK)], swc[b]).wait()


_sc_gather = functools.partial(
    pl.kernel,
    out_type=(
        jax.ShapeDtypeStruct((L * B, H), jnp.float32),
        jax.ShapeDtypeStruct((L * B, H), jnp.float32),
    ),
    mesh=plsc.VectorSubcoreMesh(core_axis_name="c", subcore_axis_name="s"),
    scratch_types=(
        [pltpu.VMEM((CHUNK, H), jnp.float32) for _ in range(4)]
        + [pltpu.VMEM((CHUNK,), jnp.int32) for _ in range(2)]
        + [pltpu.SemaphoreType.DMA for _ in range(8)]
    ),
)(_sc_gather_body)


def _sc_write_body(svh, svc, sidx, outh, outc,
                   sh0, sh1, sc0, sc1, ib0, ib1,
                   sgh0, sgh1, sgc0, sgc1, swh0, swh1, swc0, swc1):
  c = lax.axis_index("c")
  s = lax.axis_index("s")
  base0 = (c * NS + s) * RB_PER_W
  sh = (sh0, sh1)
  sc = (sc0, sc1)
  ib = (ib0, ib1)
  sgh = (sgh0, sgh1)
  sgc = (sgc0, sgc1)
  swh = (swh0, swh1)
  swc = (swc0, swc1)

  def fetch(j, b):
    base = base0 + j * CHUNK
    pltpu.sync_copy(sidx.at[pl.ds(base, CHUNK)], ib[b])
    pltpu.async_copy(svh.at[pl.ds(base, CHUNK)], sh[b], sgh[b])
    pltpu.async_copy(svc.at[pl.ds(base, CHUNK)], sc[b], sgc[b])

  fetch(0, 0)
  for t in range(N_CH):
    b = t % 2
    nb = 1 - b
    if t + 1 < N_CH:
      if t >= 1:
        pltpu.make_async_copy(sh[nb], outh.at[pl.ds(0, CHUNK)], swh[nb]).wait()
        pltpu.make_async_copy(sc[nb], outc.at[pl.ds(0, CHUNK)], swc[nb]).wait()
      fetch(t + 1, nb)
    pltpu.make_async_copy(svh.at[pl.ds(0, CHUNK)], sh[b], sgh[b]).wait()
    pltpu.async_copy(sh[b], outh.at[ib[b]], swh[b])
    pltpu.make_async_copy(svc.at[pl.ds(0, CHUNK)], sc[b], sgc[b]).wait()
    pltpu.async_copy(sc[b], outc.at[ib[b]], swc[b])
  for b in range(2):
    pltpu.make_async_copy(sh[b], outh.at[pl.ds(0, CHUNK)], swh[b]).wait()
    pltpu.make_async_copy(sc[b], outc.at[pl.ds(0, CHUNK)], swc[b]).wait()


_sc_write = functools.partial(
    pl.kernel,
    out_type=(),
    mesh=plsc.VectorSubcoreMesh(core_axis_name="c", subcore_axis_name="s"),
    scratch_types=(
        [pltpu.VMEM((CHUNK, H), jnp.float32) for _ in range(4)]
        + [pltpu.VMEM((CHUNK,), jnp.int32) for _ in range(2)]
        + [pltpu.SemaphoreType.DMA for _ in range(8)]
    ),
)(_sc_write_body)


def kernel(mem_h, mem_c, slots, h_out, c_out):
  slots = slots.astype(jnp.int32)
  iota = lax.iota(jnp.int32, B)
  ss, si = lax.sort((slots, iota), num_keys=2)
  rs = ss[::-1]
  is_start = jnp.concatenate([jnp.ones((1,), jnp.bool_), rs[1:] != rs[:-1]])
  posr = lax.cummax(jnp.where(is_start, iota, 0))
  win = si[::-1][posr][::-1]
  gidx = jnp.concatenate([slots, slots + M])
  sidx = jnp.concatenate([ss, ss + M])
  ssrc = jnp.concatenate([win, win + B])

  hin, cin = _sc_gather(
      mem_h.reshape(L * M, H), mem_c.reshape(L * M, H), gidx)
  svh, svc = _sc_gather(
      h_out.reshape(L * B, H), c_out.reshape(L * B, H), ssrc)
  outh0, outc0 = _tc_copy(mem_h.reshape(L * M, H), mem_c.reshape(L * M, H))
  rh = jax.new_ref(outh0)
  rc = jax.new_ref(outc0)
  _sc_write(svh, svc, sidx, rh, rc)
  return (hin.reshape(L, B, H), cin.reshape(L, B, H),
          rh[...].reshape(L, M, H), rc[...].reshape(L, M, H))


# speedup vs baseline: 29.3906x; 1.1030x over previous
import functools

import jax
import jax.numpy as jnp
from jax import lax
from jax.experimental import pallas as pl
from jax.experimental.pallas import tpu as pltpu
from jax.experimental.pallas import tpu_sc as plsc

L = 2
M = 100000
H = 128
B = 16384

NC = 2
NS = 16
NW = NC * NS

COPY_BLOCK = 8000
N_BLOCKS = (L * M) // COPY_BLOCK

RB_PER_W = (L * B) // NW
CHUNK = 128
N_CH = RB_PER_W // CHUNK


def _tc_copy_body(hsrc, csrc, hdst, cdst):
  hdst[...] = hsrc[...]
  cdst[...] = csrc[...]


_tc_copy = pl.pallas_call(
    _tc_copy_body,
    grid=(N_BLOCKS,),
    in_specs=[
        pl.BlockSpec((COPY_BLOCK, H), lambda i: (i, 0)),
        pl.BlockSpec((COPY_BLOCK, H), lambda i: (i, 0)),
    ],
    out_specs=[
        pl.BlockSpec((COPY_BLOCK, H), lambda i: (i, 0)),
        pl.BlockSpec((COPY_BLOCK, H), lambda i: (i, 0)),
    ],
    out_shape=[
        jax.ShapeDtypeStruct((L * M, H), jnp.float32),
        jax.ShapeDtypeStruct((L * M, H), jnp.float32),
    ],
)


def _sc_gather_body(memh, memc, idx2, hin, cin,
                    gh0, gh1, gc0, gc1, ib0, ib1,
                    sgh0, sgh1, sgc0, sgc1, swh0, swh1, swc0, swc1):
  c = lax.axis_index("c")
  s = lax.axis_index("s")
  base0 = (c * NS + s) * RB_PER_W
  gh = (gh0, gh1)
  gc = (gc0, gc1)
  ib = (ib0, ib1)
  sgh = (sgh0, sgh1)
  sgc = (sgc0, sgc1)
  swh = (swh0, swh1)
  swc = (swc0, swc1)

  def fetch(j, b):
    base = base0 + j * CHUNK
    pltpu.sync_copy(idx2.at[pl.ds(base, CHUNK)], ib[b])
    pltpu.async_copy(memh.at[ib[b]], gh[b], sgh[b])
    pltpu.async_copy(memc.at[ib[b]], gc[b], sgc[b])

  fetch(0, 0)
  for t in range(N_CH):
    b = t % 2
    nb = 1 - b
    if t + 1 < N_CH:
      if t >= 1:
        pltpu.make_async_copy(gh[nb], hin.at[pl.ds(0, CHUNK)], swh[nb]).wait()
        pltpu.make_async_copy(gc[nb], cin.at[pl.ds(0, CHUNK)], swc[nb]).wait()
      fetch(t + 1, nb)
    base = base0 + t * CHUNK
    pltpu.make_async_copy(memh.at[pl.ds(0, CHUNK)], gh[b], sgh[b]).wait()
    pltpu.async_copy(gh[b], hin.at[pl.ds(base, CHUNK)], swh[b])
    pltpu.make_async_copy(memc.at[pl.ds(0, CHUNK)], gc[b], sgc[b]).wait()
    pltpu.async_copy(gc[b], cin.at[pl.ds(base, CHUNK)], swc[b])
  for b in range(2):
    pltpu.make_async_copy(gh[b], hin.at[pl.ds(0, CHUNK)], swh[b]).wait()
    pltpu.make_async_copy(gc[b], cin.at[pl.ds(0, CHUNK)], swc[b]).wait()


_sc_gather = functools.partial(
    pl.kernel,
    out_type=(
        jax.ShapeDtypeStruct((L * B, H), jnp.float32),
        jax.ShapeDtypeStruct((L * B, H), jnp.float32),
    ),
    mesh=plsc.VectorSubcoreMesh(core_axis_name="c", subcore_axis_name="s"),
    scratch_types=(
        [pltpu.VMEM((CHUNK, H), jnp.float32) for _ in range(4)]
        + [pltpu.VMEM((CHUNK,), jnp.int32) for _ in range(2)]
        + [pltpu.SemaphoreType.DMA for _ in range(8)]
    ),
)(_sc_gather_body)


def _sc_write_body(hv, cv, sidx, ssrc, outh, outc,
                   sh0, sh1, sc0, sc1, ib0, ib1, sb0, sb1,
                   sgh0, sgh1, sgc0, sgc1, swh0, swh1, swc0, swc1):
  c = lax.axis_index("c")
  s = lax.axis_index("s")
  base0 = (c * NS + s) * RB_PER_W
  sh = (sh0, sh1)
  sc = (sc0, sc1)
  ib = (ib0, ib1)
  sb = (sb0, sb1)
  sgh = (sgh0, sgh1)
  sgc = (sgc0, sgc1)
  swh = (swh0, swh1)
  swc = (swc0, swc1)

  def fetch(j, b):
    base = base0 + j * CHUNK
    pltpu.sync_copy(sidx.at[pl.ds(base, CHUNK)], ib[b])
    pltpu.sync_copy(ssrc.at[pl.ds(base, CHUNK)], sb[b])
    pltpu.async_copy(hv.at[sb[b]], sh[b], sgh[b])
    pltpu.async_copy(cv.at[sb[b]], sc[b], sgc[b])

  fetch(0, 0)
  for t in range(N_CH):
    b = t % 2
    nb = 1 - b
    if t + 1 < N_CH:
      if t >= 1:
        pltpu.make_async_copy(sh[nb], outh.at[pl.ds(0, CHUNK)], swh[nb]).wait()
        pltpu.make_async_copy(sc[nb], outc.at[pl.ds(0, CHUNK)], swc[nb]).wait()
      fetch(t + 1, nb)
    pltpu.make_async_copy(hv.at[pl.ds(0, CHUNK)], sh[b], sgh[b]).wait()
    pltpu.async_copy(sh[b], outh.at[ib[b]], swh[b])
    pltpu.make_async_copy(cv.at[pl.ds(0, CHUNK)], sc[b], sgc[b]).wait()
    pltpu.async_copy(sc[b], outc.at[ib[b]], swc[b])
  for b in range(2):
    pltpu.make_async_copy(sh[b], outh.at[pl.ds(0, CHUNK)], swh[b]).wait()
    pltpu.make_async_copy(sc[b], outc.at[pl.ds(0, CHUNK)], swc[b]).wait()


_sc_write = functools.partial(
    pl.kernel,
    out_type=(),
    mesh=plsc.VectorSubcoreMesh(core_axis_name="c", subcore_axis_name="s"),
    scratch_types=(
        [pltpu.VMEM((CHUNK, H), jnp.float32) for _ in range(4)]
        + [pltpu.VMEM((CHUNK,), jnp.int32) for _ in range(4)]
        + [pltpu.SemaphoreType.DMA for _ in range(8)]
    ),
)(_sc_write_body)


def kernel(mem_h, mem_c, slots, h_out, c_out):
  slots = slots.astype(jnp.int32)
  iota = lax.iota(jnp.int32, B)
  ss, si = lax.sort((slots, iota), num_keys=2)
  rs = ss[::-1]
  is_start = jnp.concatenate([jnp.ones((1,), jnp.bool_), rs[1:] != rs[:-1]])
  posr = lax.cummax(jnp.where(is_start, iota, 0))
  win = si[::-1][posr][::-1]
  gidx = jnp.concatenate([slots, slots + M])
  sidx = jnp.concatenate([ss, ss + M])
  ssrc = jnp.concatenate([win, win + B])

  hin, cin = _sc_gather(
      mem_h.reshape(L * M, H), mem_c.reshape(L * M, H), gidx)
  outh0, outc0 = _tc_copy(mem_h.reshape(L * M, H), mem_c.reshape(L * M, H))
  rh = jax.new_ref(outh0)
  rc = jax.new_ref(outc0)
  _sc_write(h_out.reshape(L * B, H), c_out.reshape(L * B, H),
            sidx, ssrc, rh, rc)
  return (hin.reshape(L, B, H), cin.reshape(L, B, H),
          rh[...].reshape(L, M, H), rc[...].reshape(L, M, H))


# speedup vs baseline: 29.5448x; 1.0052x over previous
import functools

import jax
import jax.numpy as jnp
from jax import lax
from jax.experimental import pallas as pl
from jax.experimental.pallas import tpu as pltpu
from jax.experimental.pallas import tpu_sc as plsc

L = 2
M = 100000
H = 128
B = 16384

NC = 2
NS = 16
NW = NC * NS

COPY_BLOCK = 10000
N_BLOCKS = (L * M) // COPY_BLOCK

RB_PER_W = (L * B) // NW
CHUNK = 128
N_CH = RB_PER_W // CHUNK


def _tc_copy_body(hsrc, csrc, hdst, cdst):
  hdst[...] = hsrc[...]
  cdst[...] = csrc[...]


_tc_copy = pl.pallas_call(
    _tc_copy_body,
    grid=(N_BLOCKS,),
    in_specs=[
        pl.BlockSpec((COPY_BLOCK, H), lambda i: (i, 0)),
        pl.BlockSpec((COPY_BLOCK, H), lambda i: (i, 0)),
    ],
    out_specs=[
        pl.BlockSpec((COPY_BLOCK, H), lambda i: (i, 0)),
        pl.BlockSpec((COPY_BLOCK, H), lambda i: (i, 0)),
    ],
    out_shape=[
        jax.ShapeDtypeStruct((L * M, H), jnp.float32),
        jax.ShapeDtypeStruct((L * M, H), jnp.float32),
    ],
)


def _sc_gather_body(memh, memc, idx2, hin, cin,
                    gh0, gh1, gc0, gc1, ib0, ib1,
                    sgh0, sgh1, sgc0, sgc1, swh0, swh1, swc0, swc1):
  c = lax.axis_index("c")
  s = lax.axis_index("s")
  base0 = (c * NS + s) * RB_PER_W
  gh = (gh0, gh1)
  gc = (gc0, gc1)
  ib = (ib0, ib1)
  sgh = (sgh0, sgh1)
  sgc = (sgc0, sgc1)
  swh = (swh0, swh1)
  swc = (swc0, swc1)

  def fetch(j, b):
    base = base0 + j * CHUNK
    pltpu.sync_copy(idx2.at[pl.ds(base, CHUNK)], ib[b])
    pltpu.async_copy(memh.at[ib[b]], gh[b], sgh[b])
    pltpu.async_copy(memc.at[ib[b]], gc[b], sgc[b])

  fetch(0, 0)
  for t in range(N_CH):
    b = t % 2
    nb = 1 - b
    if t + 1 < N_CH:
      if t >= 1:
        pltpu.make_async_copy(gh[nb], hin.at[pl.ds(0, CHUNK)], swh[nb]).wait()
        pltpu.make_async_copy(gc[nb], cin.at[pl.ds(0, CHUNK)], swc[nb]).wait()
      fetch(t + 1, nb)
    base = base0 + t * CHUNK
    pltpu.make_async_copy(memh.at[pl.ds(0, CHUNK)], gh[b], sgh[b]).wait()
    pltpu.async_copy(gh[b], hin.at[pl.ds(base, CHUNK)], swh[b])
    pltpu.make_async_copy(memc.at[pl.ds(0, CHUNK)], gc[b], sgc[b]).wait()
    pltpu.async_copy(gc[b], cin.at[pl.ds(base, CHUNK)], swc[b])
  for b in range(2):
    pltpu.make_async_copy(gh[b], hin.at[pl.ds(0, CHUNK)], swh[b]).wait()
    pltpu.make_async_copy(gc[b], cin.at[pl.ds(0, CHUNK)], swc[b]).wait()


_sc_gather = functools.partial(
    pl.kernel,
    out_type=(
        jax.ShapeDtypeStruct((L * B, H), jnp.float32),
        jax.ShapeDtypeStruct((L * B, H), jnp.float32),
    ),
    mesh=plsc.VectorSubcoreMesh(core_axis_name="c", subcore_axis_name="s"),
    scratch_types=(
        [pltpu.VMEM((CHUNK, H), jnp.float32) for _ in range(4)]
        + [pltpu.VMEM((CHUNK,), jnp.int32) for _ in range(2)]
        + [pltpu.SemaphoreType.DMA for _ in range(8)]
    ),
)(_sc_gather_body)


def _sc_write_body(hv, cv, sidx, ssrc, outh, outc,
                   sh0, sh1, sc0, sc1, ib0, ib1, sb0, sb1,
                   sgh0, sgh1, sgc0, sgc1, swh0, swh1, swc0, swc1):
  c = lax.axis_index("c")
  s = lax.axis_index("s")
  base0 = (c * NS + s) * RB_PER_W
  sh = (sh0, sh1)
  sc = (sc0, sc1)
  ib = (ib0, ib1)
  sb = (sb0, sb1)
  sgh = (sgh0, sgh1)
  sgc = (sgc0, sgc1)
  swh = (swh0, swh1)
  swc = (swc0, swc1)

  def fetch(j, b):
    base = base0 + j * CHUNK
    pltpu.sync_copy(sidx.at[pl.ds(base, CHUNK)], ib[b])
    pltpu.sync_copy(ssrc.at[pl.ds(base, CHUNK)], sb[b])
    pltpu.async_copy(hv.at[sb[b]], sh[b], sgh[b])
    pltpu.async_copy(cv.at[sb[b]], sc[b], sgc[b])

  fetch(0, 0)
  for t in range(N_CH):
    b = t % 2
    nb = 1 - b
    if t + 1 < N_CH:
      if t >= 1:
        pltpu.make_async_copy(sh[nb], outh.at[pl.ds(0, CHUNK)], swh[nb]).wait()
        pltpu.make_async_copy(sc[nb], outc.at[pl.ds(0, CHUNK)], swc[nb]).wait()
      fetch(t + 1, nb)
    pltpu.make_async_copy(hv.at[pl.ds(0, CHUNK)], sh[b], sgh[b]).wait()
    pltpu.async_copy(sh[b], outh.at[ib[b]], swh[b])
    pltpu.make_async_copy(cv.at[pl.ds(0, CHUNK)], sc[b], sgc[b]).wait()
    pltpu.async_copy(sc[b], outc.at[ib[b]], swc[b])
  for b in range(2):
    pltpu.make_async_copy(sh[b], outh.at[pl.ds(0, CHUNK)], swh[b]).wait()
    pltpu.make_async_copy(sc[b], outc.at[pl.ds(0, CHUNK)], swc[b]).wait()


_sc_write = functools.partial(
    pl.kernel,
    out_type=(),
    mesh=plsc.VectorSubcoreMesh(core_axis_name="c", subcore_axis_name="s"),
    scratch_types=(
        [pltpu.VMEM((CHUNK, H), jnp.float32) for _ in range(4)]
        + [pltpu.VMEM((CHUNK,), jnp.int32) for _ in range(4)]
        + [pltpu.SemaphoreType.DMA for _ in range(8)]
    ),
)(_sc_write_body)


def kernel(mem_h, mem_c, slots, h_out, c_out):
  slots = slots.astype(jnp.int32)
  iota = lax.iota(jnp.int32, B)
  ss, si = lax.sort((slots, iota), num_keys=2)
  rs = ss[::-1]
  is_start = jnp.concatenate([jnp.ones((1,), jnp.bool_), rs[1:] != rs[:-1]])
  posr = lax.cummax(jnp.where(is_start, iota, 0))
  win = si[::-1][posr][::-1]
  gidx = jnp.concatenate([slots, slots + M])
  sidx = jnp.concatenate([ss, ss + M])
  ssrc = jnp.concatenate([win, win + B])

  hin, cin = _sc_gather(
      mem_h.reshape(L * M, H), mem_c.reshape(L * M, H), gidx)
  outh0, outc0 = _tc_copy(mem_h.reshape(L * M, H), mem_c.reshape(L * M, H))
  rh = jax.new_ref(outh0)
  rc = jax.new_ref(outc0)
  _sc_write(h_out.reshape(L * B, H), c_out.reshape(L * B, H),
            sidx, ssrc, rh, rc)
  return (hin.reshape(L, B, H), cin.reshape(L, B, H),
          rh[...].reshape(L, M, H), rc[...].reshape(L, M, H))
